# baseline (speedup 1.0000x reference)
import functools

import jax
import jax.numpy as jnp
from jax import lax
from jax.experimental import pallas as pl
from jax.experimental.pallas import tpu as pltpu
from jax.experimental.pallas import tpu_sc as plsc

VOCAB = 1000000
EMBED_DIM = 64
BATCH = 4096
HIST = 200
DROPOUT = 0.1

N = BATCH * HIST
NC, NS, LANES = 2, 16, 16
NW = NC * NS
N_PER_W = N // NW
CHUNK = 64
NSTEP = N_PER_W // CHUNK
NSLOT = 4
DEPTH = 2


def _sc_body(words_hbm, wview_hbm, mvals_hbm, out_hbm,
             wq_v, idx_v, sup_v, outb_v, mval_v, gsems, wsems):
    wid = lax.axis_index("s") * NC + lax.axis_index("c")
    w0 = wid * N_PER_W

    def stage_and_gather(t, b):
        base = pl.multiple_of(w0 + t * CHUNK, CHUNK)
        pltpu.sync_copy(words_hbm.at[pl.ds(base // CHUNK, 1)],
                        wq_v.at[pl.ds(b, 1)])
        for k in range(CHUNK // LANES):
            sl = pl.ds(k * LANES, LANES)
            idx_v[b, sl] = lax.shift_right_logical(wq_v[b, sl], 1)
        pltpu.async_copy(wview_hbm.at[idx_v.at[b]], sup_v.at[b], gsems[b])
        pltpu.async_copy(mvals_hbm.at[wq_v.at[b]], mval_v.at[b], gsems[b])

    def wait_gathers(b):
        pltpu.make_async_copy(
            wview_hbm.at[idx_v.at[b]], sup_v.at[b], gsems[b]).wait()
        pltpu.make_async_copy(
            mvals_hbm.at[wq_v.at[b]], mval_v.at[b], gsems[b]).wait()

    def wait_writeback(b):
        pltpu.make_async_copy(
            outb_v.at[b], out_hbm.at[pl.ds(0, CHUNK)], wsems[b]).wait()

    for t in range(DEPTH):
        stage_and_gather(t, t)

    def outer(s, carry):
        for b in range(NSLOT):
            t = s + b
            bg = (b + DEPTH) % NSLOT

            @pl.when(t + DEPTH < NSTEP)
            def _():
                @pl.when(t >= NSLOT - DEPTH)
                def _():
                    wait_writeback(bg)
                stage_and_gather(t + DEPTH, bg)

            wait_gathers(b)

            for g16 in range(CHUNK // LANES):
                r0 = g16 * LANES
                sl16 = pl.ds(r0, LANES)
                mv = mval_v[b, sl16]
                wv = wq_v[b, sl16]
                for l in range(LANES):
                    m = jnp.full((LANES,), mv[l])
                    off = (wv[l] & 1) * EMBED_DIM
                    for j in range(EMBED_DIM // LANES):
                        outb_v[b, r0 + l, pl.ds(j * LANES, LANES)] = (
                            sup_v[b, r0 + l, pl.ds(off + j * LANES, LANES)]
                            * m)

            base = pl.multiple_of(w0 + t * CHUNK, CHUNK)
            pltpu.async_copy(outb_v.at[b], out_hbm.at[pl.ds(base, CHUNK)],
                             wsems[b])
        return carry

    lax.fori_loop(0, NSTEP // NSLOT, lambda i, c: outer(i * NSLOT, c), None)

    for b in range(NSLOT):
        wait_writeback(b)


@jax.jit
def kernel(words, weight):
    mask_key = jax.random.key(42)
    keep = jax.random.bernoulli(mask_key, 1.0 - DROPOUT, (VOCAB, 1))
    mvals = (keep.astype(weight.dtype) / (1.0 - DROPOUT)).reshape(VOCAB)

    words2d = words.reshape(N // CHUNK, CHUNK)
    wview = weight.reshape(VOCAB // 2, 2 * EMBED_DIM)
    mesh = plsc.VectorSubcoreMesh(core_axis_name="c", subcore_axis_name="s")

    def body(words_hbm, wview_hbm, mvals_hbm, out_hbm,
             wq_v, idx_v, sup_v, outb_v, mval_v,
             g0, g1, g2, g3, ws0, ws1, ws2, ws3):
        _sc_body(words_hbm, wview_hbm, mvals_hbm, out_hbm,
                 wq_v, idx_v, sup_v, outb_v, mval_v,
                 (g0, g1, g2, g3), (ws0, ws1, ws2, ws3))

    out = pl.kernel(
        body,
        out_type=jax.ShapeDtypeStruct((N, EMBED_DIM), jnp.float32),
        mesh=mesh,
        scratch_types=[
            pltpu.VMEM((NSLOT, CHUNK), jnp.int32),
            pltpu.VMEM((NSLOT, CHUNK), jnp.int32),
            pltpu.VMEM((NSLOT, CHUNK, 2 * EMBED_DIM), jnp.float32),
            pltpu.VMEM((NSLOT, CHUNK, EMBED_DIM), jnp.float32),
            pltpu.VMEM((NSLOT, CHUNK), jnp.float32),
        ] + [pltpu.SemaphoreType.DMA] * (2 * NSLOT),
        compiler_params=pltpu.CompilerParams(use_tc_tiling_on_sc=True),
    )(words2d, wview, mvals)
    return out.reshape(BATCH, HIST, EMBED_DIM)


# speedup vs baseline: 1.0176x; 1.0176x over previous
import functools

import jax
import jax.numpy as jnp
from jax import lax
from jax.experimental import pallas as pl
from jax.experimental.pallas import tpu as pltpu
from jax.experimental.pallas import tpu_sc as plsc

VOCAB = 1000000
EMBED_DIM = 64
BATCH = 4096
HIST = 200
DROPOUT = 0.1

N = BATCH * HIST
NC, NS, LANES = 2, 16, 16
NW = NC * NS
N_PER_W = N // NW
CHUNK = 128
NSTEP = N_PER_W // CHUNK
NSLOT = 4
NWB = 2
DEPTH = 2


def _sc_body(words_hbm, wview_hbm, mvals_hbm, out_hbm,
             wq_v, idx_v, sup_v, outb_v, mval_v, gsems, wsems):
    wid = lax.axis_index("s") * NC + lax.axis_index("c")
    w0 = wid * N_PER_W

    def stage_and_gather(t, b):
        base = pl.multiple_of(w0 + t * CHUNK, CHUNK)
        pltpu.sync_copy(words_hbm.at[pl.ds(base // CHUNK, 1)],
                        wq_v.at[pl.ds(b, 1)])
        for k in range(CHUNK // LANES):
            sl = pl.ds(k * LANES, LANES)
            idx_v[b, sl] = lax.shift_right_logical(wq_v[b, sl], 1)
        pltpu.async_copy(wview_hbm.at[idx_v.at[b]], sup_v.at[b], gsems[b])
        pltpu.async_copy(mvals_hbm.at[wq_v.at[b]], mval_v.at[b], gsems[b])

    def wait_gathers(b):
        pltpu.make_async_copy(
            wview_hbm.at[idx_v.at[b]], sup_v.at[b], gsems[b]).wait()
        pltpu.make_async_copy(
            mvals_hbm.at[wq_v.at[b]], mval_v.at[b], gsems[b]).wait()

    def wait_writeback(w):
        pltpu.make_async_copy(
            outb_v.at[w], out_hbm.at[pl.ds(0, CHUNK)], wsems[w]).wait()

    for t in range(DEPTH):
        stage_and_gather(t, t)

    def outer(s, carry):
        for b in range(NSLOT):
            t = s + b
            bw = b % NWB

            @pl.when(t + DEPTH < NSTEP)
            def _():
                stage_and_gather(t + DEPTH, (b + DEPTH) % NSLOT)

            wait_gathers(b)

            @pl.when(t >= NWB)
            def _():
                wait_writeback(bw)

            for g16 in range(CHUNK // LANES):
                r0 = g16 * LANES
                wv = wq_v[b, pl.ds(r0, LANES)]
                mv = mval_v[b, pl.ds(r0, LANES)]
                for l in range(LANES):
                    off = (wv[l] & 1) * EMBED_DIM
                    m = jnp.full((LANES,), mv[l])
                    for j in range(EMBED_DIM // LANES):
                        outb_v[bw, r0 + l, pl.ds(j * LANES, LANES)] = (
                            sup_v[b, r0 + l, pl.ds(off + j * LANES, LANES)]
                            * m)

            base = pl.multiple_of(w0 + t * CHUNK, CHUNK)
            pltpu.async_copy(outb_v.at[bw], out_hbm.at[pl.ds(base, CHUNK)],
                             wsems[bw])
        return carry

    lax.fori_loop(0, NSTEP // NSLOT, lambda i, c: outer(i * NSLOT, c), None)

    for w in range(NWB):
        wait_writeback(w)


@jax.jit
def kernel(words, weight):
    mask_key = jax.random.key(42)
    keep = jax.random.bernoulli(mask_key, 1.0 - DROPOUT, (VOCAB, 1))
    mvals = (keep.astype(weight.dtype) / (1.0 - DROPOUT)).reshape(VOCAB)

    words2d = words.reshape(N // CHUNK, CHUNK)
    wview = weight.reshape(VOCAB // 2, 2 * EMBED_DIM)
    mesh = plsc.VectorSubcoreMesh(core_axis_name="c", subcore_axis_name="s")

    def body(words_hbm, wview_hbm, mvals_hbm, out_hbm,
             wq_v, idx_v, sup_v, outb_v, mval_v,
             g0, g1, g2, g3, ws0, ws1):
        _sc_body(words_hbm, wview_hbm, mvals_hbm, out_hbm,
                 wq_v, idx_v, sup_v, outb_v, mval_v,
                 (g0, g1, g2, g3), (ws0, ws1))

    out = pl.kernel(
        body,
        out_type=jax.ShapeDtypeStruct((N, EMBED_DIM), jnp.float32),
        mesh=mesh,
        scratch_types=[
            pltpu.VMEM((NSLOT, CHUNK), jnp.int32),
            pltpu.VMEM((NSLOT, CHUNK), jnp.int32),
            pltpu.VMEM((NSLOT, CHUNK, 2 * EMBED_DIM), jnp.float32),
            pltpu.VMEM((NWB, CHUNK, EMBED_DIM), jnp.float32),
            pltpu.VMEM((NSLOT, CHUNK), jnp.float32),
        ] + [pltpu.SemaphoreType.DMA] * (NSLOT + NWB),
        compiler_params=pltpu.CompilerParams(use_tc_tiling_on_sc=True),
    )(words2d, wview, mvals)
    return out.reshape(BATCH, HIST, EMBED_DIM)


# speedup vs baseline: 1.6920x; 1.6626x over previous
import functools

import jax
import jax.numpy as jnp
from jax import lax
from jax.experimental import pallas as pl
from jax.experimental.pallas import tpu as pltpu
from jax.experimental.pallas import tpu_sc as plsc

VOCAB = 1000000
EMBED_DIM = 64
BATCH = 4096
HIST = 200
DROPOUT = 0.1

N = BATCH * HIST
NC, NS, LANES = 2, 16, 16
NW = NC * NS
N_PER_W = N // NW
CHUNK = 128
NSTEP = N_PER_W // CHUNK
NSLOT = 4
NWB = 2
DEPTH = 2
PADW = 128


def _sc_body(words_hbm, wpad_hbm, mvals_hbm, out_hbm,
             wq_v, sup_v, outb_v, mval_v, gsems, wsems):
    wid = lax.axis_index("s") * NC + lax.axis_index("c")
    w0 = wid * N_PER_W

    def stage_and_gather(t, b):
        base = pl.multiple_of(w0 + t * CHUNK, CHUNK)
        pltpu.sync_copy(words_hbm.at[pl.ds(base // CHUNK, 1)],
                        wq_v.at[pl.ds(b, 1)])
        pltpu.async_copy(wpad_hbm.at[wq_v.at[b]], sup_v.at[b], gsems[b])
        pltpu.async_copy(mvals_hbm.at[wq_v.at[b]], mval_v.at[b], gsems[b])

    def wait_gathers(b):
        pltpu.make_async_copy(
            wpad_hbm.at[wq_v.at[b]], sup_v.at[b], gsems[b]).wait()
        pltpu.make_async_copy(
            mvals_hbm.at[wq_v.at[b]], mval_v.at[b], gsems[b]).wait()

    def wait_writeback(w):
        pltpu.make_async_copy(
            outb_v.at[w], out_hbm.at[pl.ds(0, CHUNK)], wsems[w]).wait()

    for t in range(DEPTH):
        stage_and_gather(t, t)

    def outer(s, carry):
        for b in range(NSLOT):
            t = s + b
            bw = b % NWB

            @pl.when(t + DEPTH < NSTEP)
            def _():
                stage_and_gather(t + DEPTH, (b + DEPTH) % NSLOT)

            wait_gathers(b)

            @pl.when(t >= NWB)
            def _():
                wait_writeback(bw)

            for g16 in range(CHUNK // LANES):
                r0 = g16 * LANES
                mv = mval_v[b, pl.ds(r0, LANES)]
                for l in range(LANES):
                    m = jnp.full((LANES,), mv[l])
                    for j in range(EMBED_DIM // LANES):
                        outb_v[bw, r0 + l, pl.ds(j * LANES, LANES)] = (
                            sup_v[b, r0 + l, pl.ds(j * LANES, LANES)] * m)

            base = pl.multiple_of(w0 + t * CHUNK, CHUNK)
            pltpu.async_copy(outb_v.at[bw], out_hbm.at[pl.ds(base, CHUNK)],
                             wsems[bw])
        return carry

    lax.fori_loop(0, NSTEP // NSLOT, lambda i, c: outer(i * NSLOT, c), None)

    for w in range(NWB):
        wait_writeback(w)


@jax.jit
def kernel(words, weight):
    mask_key = jax.random.key(42)
    keep = jax.random.bernoulli(mask_key, 1.0 - DROPOUT, (VOCAB, 1))
    mvals = (keep.astype(weight.dtype) / (1.0 - DROPOUT)).reshape(VOCAB)

    words2d = words.reshape(N // CHUNK, CHUNK)
    wpad = jnp.pad(weight, ((0, 0), (0, PADW - EMBED_DIM)))
    mesh = plsc.VectorSubcoreMesh(core_axis_name="c", subcore_axis_name="s")

    def body(words_hbm, wpad_hbm, mvals_hbm, out_hbm,
             wq_v, sup_v, outb_v, mval_v,
             g0, g1, g2, g3, ws0, ws1):
        _sc_body(words_hbm, wpad_hbm, mvals_hbm, out_hbm,
                 wq_v, sup_v, outb_v, mval_v,
                 (g0, g1, g2, g3), (ws0, ws1))

    out = pl.kernel(
        body,
        out_type=jax.ShapeDtypeStruct((N, EMBED_DIM), jnp.float32),
        mesh=mesh,
        scratch_types=[
            pltpu.VMEM((NSLOT, CHUNK), jnp.int32),
            pltpu.VMEM((NSLOT, CHUNK, PADW), jnp.float32),
            pltpu.VMEM((NWB, CHUNK, EMBED_DIM), jnp.float32),
            pltpu.VMEM((NSLOT, CHUNK), jnp.float32),
        ] + [pltpu.SemaphoreType.DMA] * (NSLOT + NWB),
        compiler_params=pltpu.CompilerParams(use_tc_tiling_on_sc=True),
    )(words2d, wpad, mvals)
    return out.reshape(BATCH, HIST, EMBED_DIM)
